# baseline (speedup 1.0000x reference)
import jax
import jax.numpy as jnp
from jax import lax
from jax.experimental import pallas as pl
from jax.experimental.pallas import tpu as pltpu
from jax.experimental.pallas import tpu_sc as plsc

N = 10000
D = 256
HD = 128
G = 128
C = 10
EPS = 1e-5
NP = 10240
ROWS_PER_SUB = NP // 16
EP = 163840
CH = 128
SCH = 64
NCH = 160
GS = 16
GN = NCH // GS
NBUF = 4
DCH = 40
RB = 1000
NBLK = N // RB
F32 = jnp.float32
PREC = lax.Precision.HIGHEST

_mesh = plsc.VectorSubcoreMesh(core_axis_name="c", subcore_axis_name="s")


def _sc_degree(dst32):
    def body(dst_hbm, out_hbm, acc, idxv, onesv, zerov):
        c = lax.axis_index("c")
        s = lax.axis_index("s")
        w = c * 16 + s
        pltpu.sync_copy(dst_hbm.at[w], idxv)
        for i in range(8):
            onesv[pl.ds(i * 16, 16)] = jnp.ones((16,), F32)
        for i in range(ROWS_PER_SUB // 16):
            zerov[pl.ds(i * 16, 16)] = jnp.zeros((16,), F32)
        pltpu.sync_copy(zerov, acc.at[pl.ds(s * ROWS_PER_SUB, ROWS_PER_SUB)])
        plsc.subcore_barrier()
        for j in range(DCH):
            pltpu.sync_copy(onesv, acc.at[idxv.at[j]], add=True)
        plsc.subcore_barrier()
        pltpu.sync_copy(acc.at[pl.ds(s * ROWS_PER_SUB, ROWS_PER_SUB)],
                        out_hbm.at[c, pl.ds(s * ROWS_PER_SUB, ROWS_PER_SUB)])

    return pl.kernel(
        body,
        out_type=jax.ShapeDtypeStruct((2, NP), F32),
        mesh=_mesh,
        scratch_types=[
            pltpu.VMEM_SHARED((NP,), F32),
            pltpu.VMEM((DCH, CH), jnp.int32),
            pltpu.VMEM((CH,), F32),
            pltpu.VMEM((ROWS_PER_SUB,), F32),
        ],
    )(dst32)


def _sc_scatter(u, src16, dst16):
    def body(u_hbm, src_hbm, dst_hbm, out_hbm,
             acc, srcv, dstv, buf0, buf1, buf2, buf3, zerov,
             g0, g1, g2, g3, s0, s1, s2, s3, semi):
        c = lax.axis_index("c")
        s = lax.axis_index("s")
        for i in range(16):
            for j in range(HD // 16):
                zerov[i, pl.ds(j * 16, 16)] = jnp.zeros((16,), F32)
        base = s * ROWS_PER_SUB
        for z in range(ROWS_PER_SUB // 16):
            pltpu.sync_copy(zerov, acc.at[pl.ds(base + z * 16, 16)])
        plsc.subcore_barrier()
        tbl = u_hbm.at[c]
        bufs = (buf0, buf1, buf2, buf3)
        gsem = (g0, g1, g2, g3)
        ssem = (s0, s1, s2, s3)

        def wait_gather(p):
            pltpu.make_async_copy(tbl.at[srcv.at[0, 0]], bufs[p],
                                  gsem[p]).wait()

        def wait_scatter(p):
            pltpu.make_async_copy(tbl.at[srcv.at[0, 0]], bufs[p],
                                  ssem[p]).wait()

        pltpu.sync_copy(src_hbm.at[s, 0], srcv.at[0])
        pltpu.sync_copy(dst_hbm.at[s, 0], dstv.at[0])
        pltpu.async_copy(tbl.at[srcv.at[0, 0]], buf0, g0)
        pltpu.async_copy(tbl.at[srcv.at[0, 1]], buf1, g1)

        def group(g, carry):
            slot = lax.rem(g, 2)
            nslot = lax.rem(g + 1, 2)

            @pl.when(g < GN - 1)
            def _():
                pltpu.async_copy(src_hbm.at[s, g + 1], srcv.at[nslot], semi)
                pltpu.async_copy(dst_hbm.at[s, g + 1], dstv.at[nslot], semi)

            for j in range(GS):
                cj = g * GS + j
                p = j % NBUF
                q = (j + 2) % NBUF
                wait_gather(p)
                pltpu.async_copy(bufs[p], acc.at[dstv.at[slot, j]],
                                 ssem[p], add=True)
                if j == GS - 2:
                    @pl.when(g < GN - 1)
                    def _():
                        pltpu.make_async_copy(src_hbm.at[s, 0], srcv.at[0],
                                              semi).wait()
                        pltpu.make_async_copy(dst_hbm.at[s, 0], dstv.at[0],
                                              semi).wait()

                @pl.when(cj >= 2)
                def _():
                    wait_scatter(q)

                if j < GS - 2:
                    pltpu.async_copy(tbl.at[srcv.at[slot, j + 2]],
                                     bufs[q], gsem[q])
                else:
                    @pl.when(g < GN - 1)
                    def _():
                        pltpu.async_copy(tbl.at[srcv.at[nslot, j + 2 - GS]],
                                         bufs[q], gsem[q])
            return carry

        lax.fori_loop(0, GN, group, 0)
        wait_scatter((GS - 2) % NBUF)
        wait_scatter((GS - 1) % NBUF)
        plsc.subcore_barrier()
        pltpu.sync_copy(acc.at[pl.ds(base, ROWS_PER_SUB)],
                        out_hbm.at[c, pl.ds(base, ROWS_PER_SUB)])

    return pl.kernel(
        body,
        out_type=jax.ShapeDtypeStruct((2, NP, HD), F32),
        mesh=_mesh,
        scratch_types=[
            pltpu.VMEM_SHARED((NP, HD), F32),
            pltpu.VMEM((2, GS, SCH), jnp.int32),
            pltpu.VMEM((2, GS, SCH), jnp.int32),
            pltpu.VMEM((SCH, HD), F32),
            pltpu.VMEM((SCH, HD), F32),
            pltpu.VMEM((SCH, HD), F32),
            pltpu.VMEM((SCH, HD), F32),
            pltpu.VMEM((16, HD), F32),
            pltpu.SemaphoreType.DMA,
            pltpu.SemaphoreType.DMA,
            pltpu.SemaphoreType.DMA,
            pltpu.SemaphoreType.DMA,
            pltpu.SemaphoreType.DMA,
            pltpu.SemaphoreType.DMA,
            pltpu.SemaphoreType.DMA,
            pltpu.SemaphoreType.DMA,
            pltpu.SemaphoreType.DMA,
        ],
    )(u, src16, dst16)


def _tc_first(x, deg2, W1):
    def body(x_ref, deg_ref, w_ref, u_ref, dinv_ref):
        deg = deg_ref[0] + deg_ref[1] + 1.0
        dinv = lax.rsqrt(deg)
        u = jnp.dot(x_ref[...] * dinv, w_ref[...],
                    precision=PREC, preferred_element_type=F32)
        dinv_ref[...] = dinv
        u_ref[0] = u[:, :HD]
        u_ref[1] = u[:, HD:]

    return pl.pallas_call(
        body,
        grid=(N // RB,),
        in_specs=[
            pl.BlockSpec((RB, D), lambda k: (k, 0)),
            pl.BlockSpec((2, RB, 1), lambda k: (0, k, 0)),
            pl.BlockSpec((D, D), lambda k: (0, 0)),
        ],
        out_specs=[
            pl.BlockSpec((2, RB, HD), lambda k: (0, k, 0)),
            pl.BlockSpec((RB, 1), lambda k: (k, 0)),
        ],
        out_shape=[
            jax.ShapeDtypeStruct((2, N, HD), F32),
            jax.ShapeDtypeStruct((N, 1), F32),
        ],
    )(x, deg2, W1)


def _tc_layer(sarr, u, dinv, b, g, be, Wn):
    def body(s_ref, u_ref, dinv_ref, b_ref, g_ref, be_ref, w_ref,
             un_ref, y_scr, sum_scr, sq_scr):
        k = pl.program_id(0)

        @pl.when(k == 0)
        def _():
            sum_scr[...] = jnp.zeros_like(sum_scr)
            sq_scr[...] = jnp.zeros_like(sq_scr)

        @pl.when(k < NBLK)
        def _():
            sfull = jnp.concatenate([s_ref[0], s_ref[1]], axis=1)
            ufull = jnp.concatenate([u_ref[0], u_ref[1]], axis=1)
            y = dinv_ref[...] * (sfull + ufull) + b_ref[...]
            y_scr[pl.ds(k * RB, RB), :] = y
            sum_scr[...] += jnp.sum(y, axis=0, keepdims=True)
            sq_scr[...] += jnp.sum(y * y, axis=0, keepdims=True)

        @pl.when(k >= NBLK)
        def _():
            mean = sum_scr[...] * (1.0 / N)
            var = sq_scr[...] * (1.0 / N) - mean * mean
            rstd = lax.rsqrt(var + EPS)
            y = y_scr[pl.ds((k - NBLK) * RB, RB), :]
            h = jnp.maximum((y - mean) * (rstd * g_ref[...]) + be_ref[...],
                            0.0)
            un = jnp.dot(h * dinv_ref[...], w_ref[...],
                         precision=PREC, preferred_element_type=F32)
            un_ref[0] = un[:, :HD]
            un_ref[1] = un[:, HD:]

    clamp = lambda k: jnp.minimum(k, NBLK - 1)
    wrap = lambda k: jnp.where(k < NBLK, k, k - NBLK)
    return pl.pallas_call(
        body,
        grid=(2 * NBLK,),
        in_specs=[
            pl.BlockSpec((2, RB, HD), lambda k: (0, clamp(k), 0)),
            pl.BlockSpec((2, RB, HD), lambda k: (0, clamp(k), 0)),
            pl.BlockSpec((RB, 1), lambda k: (wrap(k), 0)),
            pl.BlockSpec((1, D), lambda k: (0, 0)),
            pl.BlockSpec((1, D), lambda k: (0, 0)),
            pl.BlockSpec((1, D), lambda k: (0, 0)),
            pl.BlockSpec((D, D), lambda k: (0, 0)),
        ],
        out_specs=[
            pl.BlockSpec((2, RB, HD), lambda k: (0, jnp.maximum(k - NBLK, 0), 0)),
        ],
        out_shape=[jax.ShapeDtypeStruct((2, N, HD), F32)],
        scratch_shapes=[
            pltpu.VMEM((N, D), F32),
            pltpu.VMEM((1, D), F32),
            pltpu.VMEM((1, D), F32),
        ],
    )(sarr, u, dinv, b, g, be, Wn)[0]


def _tc_final(sarr, u, dinv, b, g, be, batch2, Wl, bl):
    def body(s_ref, u_ref, dinv_ref, b_ref, g_ref, be_ref, bat_ref,
             wl_ref, bl_ref, hG_ref, p_ref, y_scr, sum_scr, sq_scr,
             hg_scr, cnt_scr):
        k = pl.program_id(0)

        @pl.when(k == 0)
        def _():
            sum_scr[...] = jnp.zeros_like(sum_scr)
            sq_scr[...] = jnp.zeros_like(sq_scr)
            hg_scr[...] = jnp.zeros_like(hg_scr)
            cnt_scr[...] = jnp.zeros_like(cnt_scr)

        @pl.when(k < NBLK)
        def _():
            sfull = jnp.concatenate([s_ref[0], s_ref[1]], axis=1)
            ufull = jnp.concatenate([u_ref[0], u_ref[1]], axis=1)
            y = dinv_ref[...] * (sfull + ufull) + b_ref[...]
            y_scr[pl.ds(k * RB, RB), :] = y
            sum_scr[...] += jnp.sum(y, axis=0, keepdims=True)
            sq_scr[...] += jnp.sum(y * y, axis=0, keepdims=True)

        @pl.when(jnp.logical_and(k >= NBLK, k < 2 * NBLK))
        def _():
            mean = sum_scr[...] * (1.0 / N)
            var = sq_scr[...] * (1.0 / N) - mean * mean
            rstd = lax.rsqrt(var + EPS)
            y = y_scr[pl.ds((k - NBLK) * RB, RB), :]
            h = jnp.maximum((y - mean) * (rstd * g_ref[...]) + be_ref[...],
                            0.0)
            gids = lax.broadcasted_iota(jnp.int32, (RB, G), 1)
            gmat = (bat_ref[...] == gids).astype(F32)
            hg_scr[...] += lax.dot_general(gmat, h, (((0,), (0,)), ((), ())),
                                           precision=PREC,
                                           preferred_element_type=F32)
            cnt_scr[...] += lax.dot_general(gmat, jnp.ones((RB, 1), F32),
                                            (((0,), (0,)), ((), ())),
                                            precision=PREC,
                                            preferred_element_type=F32)

        @pl.when(k == 2 * NBLK)
        def _():
            hG = hg_scr[...] / jnp.maximum(cnt_scr[...], 1.0)
            logits = jnp.dot(hG, wl_ref[...], precision=PREC,
                             preferred_element_type=F32) + bl_ref[...]
            m = jnp.max(logits, axis=1, keepdims=True)
            e = jnp.exp(logits - m)
            hG_ref[...] = hG
            p_ref[...] = e / jnp.sum(e, axis=1, keepdims=True)

    clamp = lambda k: jnp.minimum(k, NBLK - 1)
    mid = lambda k: jnp.clip(k - NBLK, 0, NBLK - 1)
    return pl.pallas_call(
        body,
        grid=(2 * NBLK + 1,),
        in_specs=[
            pl.BlockSpec((2, RB, HD), lambda k: (0, clamp(k), 0)),
            pl.BlockSpec((2, RB, HD), lambda k: (0, clamp(k), 0)),
            pl.BlockSpec((RB, 1), lambda k: (clamp(k), 0)),
            pl.BlockSpec((1, D), lambda k: (0, 0)),
            pl.BlockSpec((1, D), lambda k: (0, 0)),
            pl.BlockSpec((1, D), lambda k: (0, 0)),
            pl.BlockSpec((RB, 1), lambda k: (mid(k), 0)),
            pl.BlockSpec((D, C), lambda k: (0, 0)),
            pl.BlockSpec((1, C), lambda k: (0, 0)),
        ],
        out_specs=[
            pl.BlockSpec((G, D), lambda k: (0, 0)),
            pl.BlockSpec((G, C), lambda k: (0, 0)),
        ],
        out_shape=[
            jax.ShapeDtypeStruct((G, D), F32),
            jax.ShapeDtypeStruct((G, C), F32),
        ],
        scratch_shapes=[
            pltpu.VMEM((N, D), F32),
            pltpu.VMEM((1, D), F32),
            pltpu.VMEM((1, D), F32),
            pltpu.VMEM((G, D), F32),
            pltpu.VMEM((G, 1), F32),
        ],
    )(sarr, u, dinv, b, g, be, batch2, Wl, bl)


def kernel(x, edge_index, batch,
           W1, b1, g1, be1, W2, b2, g2, be2, W3, b3, g3, be3, Wl, bl):
    src = edge_index[0]
    dst = edge_index[1]
    e = src.shape[0]
    pad = EP - e
    ar = jnp.arange(pad, dtype=jnp.int32)
    padsrc = (ar * 997) % N
    paddst = N + (ar % (NP - N))
    srcp = jnp.concatenate([src, padsrc]).reshape(16, GN, GS, SCH)
    dstp = jnp.concatenate([dst, paddst])
    dst16 = dstp.reshape(16, GN, GS, SCH)
    dst32 = dstp.reshape(32, DCH, CH)

    deg2 = _sc_degree(dst32)
    u1, dinv = _tc_first(x, deg2[:, :, None], W1)
    s1 = _sc_scatter(u1, srcp, dst16)
    u2 = _tc_layer(s1, u1, dinv, b1.reshape(1, D), g1.reshape(1, D),
                   be1.reshape(1, D), W2)
    s2 = _sc_scatter(u2, srcp, dst16)
    u3 = _tc_layer(s2, u2, dinv, b2.reshape(1, D), g2.reshape(1, D),
                   be2.reshape(1, D), W3)
    s3 = _sc_scatter(u3, srcp, dst16)
    hG, probs = _tc_final(s3, u3, dinv, b3.reshape(1, D), g3.reshape(1, D),
                          be3.reshape(1, D), batch[:, None], Wl,
                          bl.reshape(1, C))
    return (hG, probs)


# speedup vs baseline: 1.1497x; 1.1497x over previous
import jax
import jax.numpy as jnp
from jax import lax
from jax.experimental import pallas as pl
from jax.experimental.pallas import tpu as pltpu
from jax.experimental.pallas import tpu_sc as plsc

N = 10000
D = 256
HD = 128
G = 128
C = 10
EPS = 1e-5
NP = 10240
ROWS_PER_SUB = NP // 16
EP = 163840
CH = 128
SCH = 128
NCH = 80
GS = 16
GN = NCH // GS
DCH = 40
RB = 1000
NBLK = N // RB
F32 = jnp.float32
PREC = lax.Precision.DEFAULT

_mesh = plsc.VectorSubcoreMesh(core_axis_name="c", subcore_axis_name="s")


def _sc_degree(dst32):
    def body(dst_hbm, out_hbm, acc, idxv, onesv, zerov):
        c = lax.axis_index("c")
        s = lax.axis_index("s")
        w = c * 16 + s
        pltpu.sync_copy(dst_hbm.at[w], idxv)
        for i in range(8):
            onesv[pl.ds(i * 16, 16)] = jnp.ones((16,), F32)
        for i in range(ROWS_PER_SUB // 16):
            zerov[pl.ds(i * 16, 16)] = jnp.zeros((16,), F32)
        pltpu.sync_copy(zerov, acc.at[pl.ds(s * ROWS_PER_SUB, ROWS_PER_SUB)])
        plsc.subcore_barrier()
        for j in range(DCH):
            pltpu.sync_copy(onesv, acc.at[idxv.at[j]], add=True)
        plsc.subcore_barrier()
        pltpu.sync_copy(acc.at[pl.ds(s * ROWS_PER_SUB, ROWS_PER_SUB)],
                        out_hbm.at[c, pl.ds(s * ROWS_PER_SUB, ROWS_PER_SUB)])

    return pl.kernel(
        body,
        out_type=jax.ShapeDtypeStruct((2, NP), F32),
        mesh=_mesh,
        scratch_types=[
            pltpu.VMEM_SHARED((NP,), F32),
            pltpu.VMEM((DCH, CH), jnp.int32),
            pltpu.VMEM((CH,), F32),
            pltpu.VMEM((ROWS_PER_SUB,), F32),
        ],
    )(dst32)


def _sc_scatter(u, src16, dst16):
    def body(u_hbm, src_hbm, dst_hbm, out_hbm,
             acc, srcv, dstv, buf0, buf1, zerov,
             g0, g1, s0, s1, semi):
        c = lax.axis_index("c")
        s = lax.axis_index("s")
        for i in range(32):
            for j in range(HD // 16):
                zerov[i, pl.ds(j * 16, 16)] = jnp.zeros((16,), F32)
        base = s * ROWS_PER_SUB
        nz = ROWS_PER_SUB // 32
        for z in range(nz):
            pltpu.async_copy(zerov, acc.at[pl.ds(base + z * 32, 32)], semi)
        for z in range(nz):
            pltpu.make_async_copy(zerov, acc.at[pl.ds(base, 32)], semi).wait()
        plsc.subcore_barrier()
        tbl = u_hbm.at[c]
        bufs = (buf0, buf1)
        gsem = (g0, g1)
        ssem = (s0, s1)

        def wait_gather(p):
            pltpu.make_async_copy(tbl.at[srcv.at[0, 0]], bufs[p],
                                  gsem[p]).wait()

        def wait_scatter(p):
            pltpu.make_async_copy(tbl.at[srcv.at[0, 0]], bufs[p],
                                  ssem[p]).wait()

        pltpu.sync_copy(src_hbm.at[s, 0], srcv.at[0])
        pltpu.sync_copy(dst_hbm.at[s, 0], dstv.at[0])
        pltpu.async_copy(tbl.at[srcv.at[0, 0]], buf0, g0)
        pltpu.async_copy(tbl.at[srcv.at[0, 1]], buf1, g1)

        def group(g, carry):
            slot = lax.rem(g, 2)
            nslot = lax.rem(g + 1, 2)

            @pl.when(g < GN - 1)
            def _():
                pltpu.async_copy(src_hbm.at[s, g + 1], srcv.at[nslot], semi)
                pltpu.async_copy(dst_hbm.at[s, g + 1], dstv.at[nslot], semi)

            for j in range(GS):
                p = j % 2
                wait_gather(p)
                pltpu.sync_copy(bufs[p], acc.at[dstv.at[slot, j]], add=True)
                if j == GS - 2:
                    @pl.when(g < GN - 1)
                    def _():
                        pltpu.make_async_copy(src_hbm.at[s, 0], srcv.at[0],
                                              semi).wait()
                        pltpu.make_async_copy(dst_hbm.at[s, 0], dstv.at[0],
                                              semi).wait()
                if j < GS - 2:
                    pltpu.async_copy(tbl.at[srcv.at[slot, j + 2]],
                                     bufs[p], gsem[p])
                else:
                    @pl.when(g < GN - 1)
                    def _():
                        pltpu.async_copy(tbl.at[srcv.at[nslot, j + 2 - GS]],
                                         bufs[p], gsem[p])
            return carry

        lax.fori_loop(0, GN, group, 0)
        plsc.subcore_barrier()
        pltpu.sync_copy(acc.at[pl.ds(base, ROWS_PER_SUB)],
                        out_hbm.at[c, pl.ds(base, ROWS_PER_SUB)])

    return pl.kernel(
        body,
        out_type=jax.ShapeDtypeStruct((2, NP, HD), F32),
        mesh=_mesh,
        scratch_types=[
            pltpu.VMEM_SHARED((NP, HD), F32),
            pltpu.VMEM((2, GS, SCH), jnp.int32),
            pltpu.VMEM((2, GS, SCH), jnp.int32),
            pltpu.VMEM((SCH, HD), F32),
            pltpu.VMEM((SCH, HD), F32),
            pltpu.VMEM((32, HD), F32),
            pltpu.SemaphoreType.DMA,
            pltpu.SemaphoreType.DMA,
            pltpu.SemaphoreType.DMA,
            pltpu.SemaphoreType.DMA,
            pltpu.SemaphoreType.DMA,
        ],
    )(u, src16, dst16)


def _tc_first(x, deg2, W1):
    def body(x_ref, deg_ref, w_ref, u_ref, dinv_ref):
        deg = deg_ref[0] + deg_ref[1] + 1.0
        dinv = lax.rsqrt(deg)
        u = jnp.dot(x_ref[...] * dinv, w_ref[...],
                    precision=PREC, preferred_element_type=F32)
        dinv_ref[...] = dinv
        u_ref[0] = u[:, :HD]
        u_ref[1] = u[:, HD:]

    return pl.pallas_call(
        body,
        grid=(N // RB,),
        in_specs=[
            pl.BlockSpec((RB, D), lambda k: (k, 0)),
            pl.BlockSpec((2, RB, 1), lambda k: (0, k, 0)),
            pl.BlockSpec((D, D), lambda k: (0, 0)),
        ],
        out_specs=[
            pl.BlockSpec((2, RB, HD), lambda k: (0, k, 0)),
            pl.BlockSpec((RB, 1), lambda k: (k, 0)),
        ],
        out_shape=[
            jax.ShapeDtypeStruct((2, N, HD), F32),
            jax.ShapeDtypeStruct((N, 1), F32),
        ],
    )(x, deg2, W1)


def _tc_layer(sarr, u, dinv, b, g, be, Wn):
    def body(s_ref, u_ref, dinv_ref, b_ref, g_ref, be_ref, w_ref,
             un_ref, y_scr, sum_scr, sq_scr):
        k = pl.program_id(0)

        @pl.when(k == 0)
        def _():
            sum_scr[...] = jnp.zeros_like(sum_scr)
            sq_scr[...] = jnp.zeros_like(sq_scr)

        @pl.when(k < NBLK)
        def _():
            sfull = jnp.concatenate([s_ref[0], s_ref[1]], axis=1)
            ufull = jnp.concatenate([u_ref[0], u_ref[1]], axis=1)
            y = dinv_ref[...] * (sfull + ufull) + b_ref[...]
            y_scr[pl.ds(k * RB, RB), :] = y
            sum_scr[...] += jnp.sum(y, axis=0, keepdims=True)
            sq_scr[...] += jnp.sum(y * y, axis=0, keepdims=True)

        @pl.when(k >= NBLK)
        def _():
            mean = sum_scr[...] * (1.0 / N)
            var = sq_scr[...] * (1.0 / N) - mean * mean
            rstd = lax.rsqrt(var + EPS)
            y = y_scr[pl.ds((k - NBLK) * RB, RB), :]
            h = jnp.maximum((y - mean) * (rstd * g_ref[...]) + be_ref[...],
                            0.0)
            un = jnp.dot(h * dinv_ref[...], w_ref[...],
                         precision=PREC, preferred_element_type=F32)
            un_ref[0] = un[:, :HD]
            un_ref[1] = un[:, HD:]

    clamp = lambda k: jnp.minimum(k, NBLK - 1)
    wrap = lambda k: jnp.where(k < NBLK, k, k - NBLK)
    return pl.pallas_call(
        body,
        grid=(2 * NBLK,),
        in_specs=[
            pl.BlockSpec((2, RB, HD), lambda k: (0, clamp(k), 0)),
            pl.BlockSpec((2, RB, HD), lambda k: (0, clamp(k), 0)),
            pl.BlockSpec((RB, 1), lambda k: (wrap(k), 0)),
            pl.BlockSpec((1, D), lambda k: (0, 0)),
            pl.BlockSpec((1, D), lambda k: (0, 0)),
            pl.BlockSpec((1, D), lambda k: (0, 0)),
            pl.BlockSpec((D, D), lambda k: (0, 0)),
        ],
        out_specs=[
            pl.BlockSpec((2, RB, HD), lambda k: (0, jnp.maximum(k - NBLK, 0), 0)),
        ],
        out_shape=[jax.ShapeDtypeStruct((2, N, HD), F32)],
        scratch_shapes=[
            pltpu.VMEM((N, D), F32),
            pltpu.VMEM((1, D), F32),
            pltpu.VMEM((1, D), F32),
        ],
    )(sarr, u, dinv, b, g, be, Wn)[0]


def _tc_final(sarr, u, dinv, b, g, be, batch2, Wl, bl):
    def body(s_ref, u_ref, dinv_ref, b_ref, g_ref, be_ref, bat_ref,
             wl_ref, bl_ref, hG_ref, p_ref, y_scr, sum_scr, sq_scr,
             hg_scr, cnt_scr):
        k = pl.program_id(0)

        @pl.when(k == 0)
        def _():
            sum_scr[...] = jnp.zeros_like(sum_scr)
            sq_scr[...] = jnp.zeros_like(sq_scr)
            hg_scr[...] = jnp.zeros_like(hg_scr)
            cnt_scr[...] = jnp.zeros_like(cnt_scr)

        @pl.when(k < NBLK)
        def _():
            sfull = jnp.concatenate([s_ref[0], s_ref[1]], axis=1)
            ufull = jnp.concatenate([u_ref[0], u_ref[1]], axis=1)
            y = dinv_ref[...] * (sfull + ufull) + b_ref[...]
            y_scr[pl.ds(k * RB, RB), :] = y
            sum_scr[...] += jnp.sum(y, axis=0, keepdims=True)
            sq_scr[...] += jnp.sum(y * y, axis=0, keepdims=True)

        @pl.when(jnp.logical_and(k >= NBLK, k < 2 * NBLK))
        def _():
            mean = sum_scr[...] * (1.0 / N)
            var = sq_scr[...] * (1.0 / N) - mean * mean
            rstd = lax.rsqrt(var + EPS)
            y = y_scr[pl.ds((k - NBLK) * RB, RB), :]
            h = jnp.maximum((y - mean) * (rstd * g_ref[...]) + be_ref[...],
                            0.0)
            gids = lax.broadcasted_iota(jnp.int32, (RB, G), 1)
            gmat = (bat_ref[...] == gids).astype(F32)
            hg_scr[...] += lax.dot_general(gmat, h, (((0,), (0,)), ((), ())),
                                           precision=PREC,
                                           preferred_element_type=F32)
            cnt_scr[...] += lax.dot_general(gmat, jnp.ones((RB, 1), F32),
                                            (((0,), (0,)), ((), ())),
                                            precision=PREC,
                                            preferred_element_type=F32)

        @pl.when(k == 2 * NBLK)
        def _():
            hG = hg_scr[...] / jnp.maximum(cnt_scr[...], 1.0)
            logits = jnp.dot(hG, wl_ref[...], precision=PREC,
                             preferred_element_type=F32) + bl_ref[...]
            m = jnp.max(logits, axis=1, keepdims=True)
            e = jnp.exp(logits - m)
            hG_ref[...] = hG
            p_ref[...] = e / jnp.sum(e, axis=1, keepdims=True)

    clamp = lambda k: jnp.minimum(k, NBLK - 1)
    mid = lambda k: jnp.clip(k - NBLK, 0, NBLK - 1)
    return pl.pallas_call(
        body,
        grid=(2 * NBLK + 1,),
        in_specs=[
            pl.BlockSpec((2, RB, HD), lambda k: (0, clamp(k), 0)),
            pl.BlockSpec((2, RB, HD), lambda k: (0, clamp(k), 0)),
            pl.BlockSpec((RB, 1), lambda k: (clamp(k), 0)),
            pl.BlockSpec((1, D), lambda k: (0, 0)),
            pl.BlockSpec((1, D), lambda k: (0, 0)),
            pl.BlockSpec((1, D), lambda k: (0, 0)),
            pl.BlockSpec((RB, 1), lambda k: (mid(k), 0)),
            pl.BlockSpec((D, C), lambda k: (0, 0)),
            pl.BlockSpec((1, C), lambda k: (0, 0)),
        ],
        out_specs=[
            pl.BlockSpec((G, D), lambda k: (0, 0)),
            pl.BlockSpec((G, C), lambda k: (0, 0)),
        ],
        out_shape=[
            jax.ShapeDtypeStruct((G, D), F32),
            jax.ShapeDtypeStruct((G, C), F32),
        ],
        scratch_shapes=[
            pltpu.VMEM((N, D), F32),
            pltpu.VMEM((1, D), F32),
            pltpu.VMEM((1, D), F32),
            pltpu.VMEM((G, D), F32),
            pltpu.VMEM((G, 1), F32),
        ],
    )(sarr, u, dinv, b, g, be, batch2, Wl, bl)


def kernel(x, edge_index, batch,
           W1, b1, g1, be1, W2, b2, g2, be2, W3, b3, g3, be3, Wl, bl):
    src = edge_index[0]
    dst = edge_index[1]
    e = src.shape[0]
    pad = EP - e
    ar = jnp.arange(pad, dtype=jnp.int32)
    padsrc = (ar * 997) % N
    paddst = N + (ar % (NP - N))
    srcp = jnp.concatenate([src, padsrc]).reshape(16, GN, GS, SCH)
    dstp = jnp.concatenate([dst, paddst])
    dst16 = dstp.reshape(16, GN, GS, SCH)
    dst32 = dstp.reshape(32, DCH, CH)

    deg2 = _sc_degree(dst32)
    u1, dinv = _tc_first(x, deg2[:, :, None], W1)
    s1 = _sc_scatter(u1, srcp, dst16)
    u2 = _tc_layer(s1, u1, dinv, b1.reshape(1, D), g1.reshape(1, D),
                   be1.reshape(1, D), W2)
    s2 = _sc_scatter(u2, srcp, dst16)
    u3 = _tc_layer(s2, u2, dinv, b2.reshape(1, D), g2.reshape(1, D),
                   be2.reshape(1, D), W3)
    s3 = _sc_scatter(u3, srcp, dst16)
    hG, probs = _tc_final(s3, u3, dinv, b3.reshape(1, D), g3.reshape(1, D),
                          be3.reshape(1, D), batch[:, None], Wl,
                          bl.reshape(1, C))
    return (hG, probs)
